# baseline (speedup 1.0000x reference)
import jax
import jax.numpy as jnp
from jax.experimental import pallas as pl
from jax.experimental.pallas import tpu as pltpu

B, H, W, D, K = 8, 28, 28, 384, 20000
QB = H * W
QP = 896
KB = 4096
CHUNK = 512
K_PAD = ((K + KB - 1) // KB) * KB
NK = K_PAD // KB
D_AUG = 400


def _knn_kernel(m_ref, q_ref, patch_ref, img_ref, acc_ref):
    k = pl.program_id(1)
    q = q_ref[...]
    red = None
    for c in range(KB // CHUNK):
        t = jax.lax.dot_general(
            m_ref[c * CHUNK:(c + 1) * CHUNK, :], q,
            dimension_numbers=(((1,), (0,)), ((), ())),
            preferred_element_type=jnp.float32)
        for r in range(CHUNK // 8):
            sl = t[r * 8:(r + 1) * 8, :]
            red = sl if red is None else jnp.minimum(red, sl)

    @pl.when(k == 0)
    def _init():
        acc_ref[...] = red

    @pl.when(k > 0)
    def _acc():
        acc_ref[...] = jnp.minimum(acc_ref[...], red)

    @pl.when(k == NK - 1)
    def _finish():
        q32 = q.astype(jnp.float32)
        q_sq = 0.25 * (jnp.sum(q32 * q32, axis=0) - 2.0)
        tmin = jnp.min(acc_ref[...], axis=0)
        nn = jnp.sqrt(jnp.maximum(q_sq + tmin, 1e-12))
        patch_ref[0, 0, :] = nn
        img_ref[0, 0, :] = jnp.full((128,), jnp.max(nn), dtype=jnp.float32)


@jax.jit
def kernel(queries, memory_bank):
    qr = queries.reshape(B, QB, D)
    qp = jnp.concatenate([qr, qr[:, :QP - QB]], axis=1)
    q2 = (-2.0 * qp.reshape(B * QP, D)).astype(jnp.bfloat16)
    q_t = jnp.concatenate(
        [q2.T,
         jnp.ones((2, B * QP), jnp.bfloat16),
         jnp.zeros((D_AUG - D - 2, B * QP), jnp.bfloat16)], axis=0)

    m_sq = jnp.sum(memory_bank * memory_bank, axis=1)
    msq_hi = m_sq.astype(jnp.bfloat16)
    msq_lo = (m_sq - msq_hi.astype(jnp.float32)).astype(jnp.bfloat16)
    m_real = jnp.concatenate(
        [memory_bank.astype(jnp.bfloat16),
         msq_hi[:, None], msq_lo[:, None],
         jnp.zeros((K, D_AUG - D - 2), jnp.bfloat16)], axis=1)
    pad_blk = jnp.zeros((K_PAD - K, D_AUG), jnp.bfloat16).at[:, D].set(1e10)
    m_aug = jnp.concatenate([m_real, pad_blk], axis=0)

    patch, img = pl.pallas_call(
        _knn_kernel,
        grid=(B, NK),
        in_specs=[
            pl.BlockSpec((KB, D_AUG), lambda i, k: (k, 0)),
            pl.BlockSpec((D_AUG, QP), lambda i, k: (0, i)),
        ],
        out_specs=[
            pl.BlockSpec((1, 1, QP), lambda i, k: (i, 0, 0)),
            pl.BlockSpec((1, 1, 128), lambda i, k: (i, 0, 0)),
        ],
        out_shape=[
            jax.ShapeDtypeStruct((B, 1, QP), jnp.float32),
            jax.ShapeDtypeStruct((B, 1, 128), jnp.float32),
        ],
        scratch_shapes=[pltpu.VMEM((8, QP), jnp.float32)],
    )(m_aug, q_t)
    return patch[:, 0, :QB].reshape(B, H, W), img[:, 0, 0]


# speedup vs baseline: 1.3715x; 1.3715x over previous
import jax
import jax.numpy as jnp
from jax.experimental import pallas as pl
from jax.experimental.pallas import tpu as pltpu

B, H, W, D, K = 8, 28, 28, 384, 20000
QB = H * W
KB = 4096
CHUNK = 1024
K_PAD = ((K + KB - 1) // KB) * KB
NK = K_PAD // KB
D_AUG = 400


def _knn_kernel(q_ref, m_ref, patch_ref, img_ref, acc_ref):
    k = pl.program_id(1)
    q = q_ref[...]
    reds = [None, None]
    for c in range(KB // CHUNK):
        t = jax.lax.dot_general(
            q, m_ref[:, c * CHUNK:(c + 1) * CHUNK],
            dimension_numbers=(((1,), (0,)), ((), ())),
            preferred_element_type=jnp.float32)
        h = CHUNK
        while h > 128:
            h //= 2
            t = jnp.minimum(t[:, :h], t[:, h:2 * h])
        p = c % 2
        reds[p] = t if reds[p] is None else jnp.minimum(reds[p], t)
    red = jnp.minimum(reds[0], reds[1])

    @pl.when(k == 0)
    def _init():
        acc_ref[...] = red

    @pl.when(k > 0)
    def _acc():
        acc_ref[...] = jnp.minimum(acc_ref[...], red)

    @pl.when(k == NK - 1)
    def _finish():
        q32 = q.astype(jnp.float32)
        q_sq = 0.25 * (jnp.sum(q32 * q32, axis=1) - 2.0)
        tmin = jnp.min(acc_ref[...], axis=1)
        nn = jnp.sqrt(jnp.maximum(q_sq + tmin, 1e-12))
        patch_ref[0, 0, :] = nn
        img_ref[0, 0, :] = jnp.full((128,), jnp.max(nn), dtype=jnp.float32)


@jax.jit
def kernel(queries, memory_bank):
    qn = queries.reshape(B * QB, D)
    q_aug = jnp.concatenate(
        [(-2.0 * qn).astype(jnp.bfloat16),
         jnp.ones((B * QB, 2), jnp.bfloat16),
         jnp.zeros((B * QB, D_AUG - D - 2), jnp.bfloat16)], axis=1)

    m_sq = jnp.sum(memory_bank * memory_bank, axis=1)
    msq_hi = m_sq.astype(jnp.bfloat16)
    msq_lo = (m_sq - msq_hi.astype(jnp.float32)).astype(jnp.bfloat16)
    m_real = jnp.concatenate(
        [memory_bank.astype(jnp.bfloat16).T,
         msq_hi[None, :], msq_lo[None, :],
         jnp.zeros((D_AUG - D - 2, K), jnp.bfloat16)], axis=0)
    pad_blk = jnp.zeros((D_AUG, K_PAD - K), jnp.bfloat16).at[D, :].set(1e10)
    m_aug = jnp.concatenate([m_real, pad_blk], axis=1)

    patch, img = pl.pallas_call(
        _knn_kernel,
        grid=(B, NK),
        in_specs=[
            pl.BlockSpec((QB, D_AUG), lambda i, k: (i, 0)),
            pl.BlockSpec((D_AUG, KB), lambda i, k: (0, k)),
        ],
        out_specs=[
            pl.BlockSpec((1, 1, QB), lambda i, k: (i, 0, 0)),
            pl.BlockSpec((1, 1, 128), lambda i, k: (i, 0, 0)),
        ],
        out_shape=[
            jax.ShapeDtypeStruct((B, 1, QB), jnp.float32),
            jax.ShapeDtypeStruct((B, 1, 128), jnp.float32),
        ],
        scratch_shapes=[pltpu.VMEM((QB, 128), jnp.float32)],
    )(q_aug, m_aug)
    return patch.reshape(B, H, W), img[:, 0, 0]


# speedup vs baseline: 1.4172x; 1.0333x over previous
import jax
import jax.numpy as jnp
from jax.experimental import pallas as pl

B, H, W, D, K = 8, 28, 28, 384, 20000
QB = H * W
CHUNK = 1024
K_PAD = ((K + CHUNK - 1) // CHUNK) * CHUNK
D_AUG = 400


def _knn_kernel(q_ref, m_ref, patch_ref, img_ref):
    q = q_ref[...]
    reds = [None, None]
    for c in range(K_PAD // CHUNK):
        t = jax.lax.dot_general(
            q, m_ref[:, c * CHUNK:(c + 1) * CHUNK],
            dimension_numbers=(((1,), (0,)), ((), ())),
            preferred_element_type=jnp.float32)
        h = CHUNK
        while h > 128:
            h //= 2
            t = jnp.minimum(t[:, :h], t[:, h:2 * h])
        p = c % 2
        reds[p] = t if reds[p] is None else jnp.minimum(reds[p], t)
    red = jnp.minimum(reds[0], reds[1])

    q32 = q.astype(jnp.float32)
    q_sq = 0.25 * (jnp.sum(q32 * q32, axis=1) - 2.0)
    tmin = jnp.min(red, axis=1)
    nn = jnp.sqrt(jnp.maximum(q_sq + tmin, 1e-12))
    patch_ref[0, 0, :] = nn
    img_ref[0, 0, :] = jnp.full((128,), jnp.max(nn), dtype=jnp.float32)


@jax.jit
def kernel(queries, memory_bank):
    qn = queries.reshape(B * QB, D)
    q_aug = jnp.concatenate(
        [(-2.0 * qn).astype(jnp.bfloat16),
         jnp.ones((B * QB, 2), jnp.bfloat16),
         jnp.zeros((B * QB, D_AUG - D - 2), jnp.bfloat16)], axis=1)

    m_sq = jnp.sum(memory_bank * memory_bank, axis=1)
    msq_hi = m_sq.astype(jnp.bfloat16)
    msq_lo = (m_sq - msq_hi.astype(jnp.float32)).astype(jnp.bfloat16)
    m_real = jnp.concatenate(
        [memory_bank.astype(jnp.bfloat16).T,
         msq_hi[None, :], msq_lo[None, :],
         jnp.zeros((D_AUG - D - 2, K), jnp.bfloat16)], axis=0)
    pad_blk = jnp.zeros((D_AUG, K_PAD - K), jnp.bfloat16).at[D, :].set(1e10)
    m_aug = jnp.concatenate([m_real, pad_blk], axis=1)

    patch, img = pl.pallas_call(
        _knn_kernel,
        grid=(B,),
        in_specs=[
            pl.BlockSpec((QB, D_AUG), lambda i: (i, 0)),
            pl.BlockSpec((D_AUG, K_PAD), lambda i: (0, 0)),
        ],
        out_specs=[
            pl.BlockSpec((1, 1, QB), lambda i: (i, 0, 0)),
            pl.BlockSpec((1, 1, 128), lambda i: (i, 0, 0)),
        ],
        out_shape=[
            jax.ShapeDtypeStruct((B, 1, QB), jnp.float32),
            jax.ShapeDtypeStruct((B, 1, 128), jnp.float32),
        ],
    )(q_aug, m_aug)
    return patch.reshape(B, H, W), img[:, 0, 0]
